# Initial kernel scaffold; baseline (speedup 1.0000x reference)
import jax
import jax.numpy as jnp
from jax.experimental import pallas as pl


def kernel(x_categorical, x_numerical, tables, bn_gamma, bn_beta, bn_mean, bn_var, W1, b1, W2, b2):
    raise NotImplementedError("write your pallas kernel here")


# speedup vs baseline: 8.0715x; 8.0715x over previous
import functools

import jax
import jax.numpy as jnp
from jax import lax
from jax.experimental import pallas as pl
from jax.experimental.pallas import tpu as pltpu
from jax.experimental.pallas import tpu_sc as plsc

B = 16384
N_FIELDS = 26
VOCAB = 100000
EMB = 32
NUM = 13
H1 = 256
H2 = 128
EMB_DIM = N_FIELDS * EMB
NUM_PAD = 64

N_ROWS = B * N_FIELDS
NW = 32
PER_W = N_ROWS // NW
CHUNK = 1664
NCHUNK = PER_W // CHUNK

_mesh = plsc.VectorSubcoreMesh(core_axis_name="c", subcore_axis_name="s")


@functools.partial(
    pl.kernel,
    mesh=_mesh,
    compiler_params=pltpu.CompilerParams(use_tc_tiling_on_sc=False),
    out_type=jax.ShapeDtypeStruct((N_ROWS, EMB), jnp.float32),
    scratch_types=[
        pltpu.VMEM((CHUNK,), jnp.int32),
        pltpu.VMEM((CHUNK, EMB), jnp.float32),
        pltpu.SemaphoreType.DMA,
    ],
)
def _sc_gather(idx_hbm, table_hbm, out_hbm, idx_v, rows_v, sem):
    wid = lax.axis_index("s") * 2 + lax.axis_index("c")
    base = wid * PER_W

    def body(c, carry):
        off = base + c * CHUNK
        pltpu.sync_copy(idx_hbm.at[pl.ds(off, CHUNK)], idx_v)
        pltpu.async_copy(table_hbm.at[idx_v], rows_v, sem).wait()
        pltpu.sync_copy(rows_v, out_hbm.at[pl.ds(off, CHUNK)])
        return carry

    lax.fori_loop(0, NCHUNK, body, 0)


BM = 2048


def _mlp_body(emb_ref, num_ref, w1e_ref, w1n_ref, b1_ref, w2_ref, b2_ref, out_ref):
    h = jnp.dot(emb_ref[...], w1e_ref[...], preferred_element_type=jnp.float32)
    h = h + jnp.dot(num_ref[...], w1n_ref[...], preferred_element_type=jnp.float32)
    h = jnp.maximum(h + b1_ref[...], 0.0)
    o = jnp.dot(h, w2_ref[...], preferred_element_type=jnp.float32) + b2_ref[...]
    out_ref[...] = jnp.maximum(o, 0.0)


_mlp = pl.pallas_call(
    _mlp_body,
    grid=(B // BM,),
    in_specs=[
        pl.BlockSpec((BM, EMB_DIM), lambda i: (i, 0)),
        pl.BlockSpec((BM, NUM_PAD), lambda i: (i, 0)),
        pl.BlockSpec((EMB_DIM, H1), lambda i: (0, 0)),
        pl.BlockSpec((NUM_PAD, H1), lambda i: (0, 0)),
        pl.BlockSpec((1, H1), lambda i: (0, 0)),
        pl.BlockSpec((H1, H2), lambda i: (0, 0)),
        pl.BlockSpec((1, H2), lambda i: (0, 0)),
    ],
    out_specs=pl.BlockSpec((BM, H2), lambda i: (i, 0)),
    out_shape=jax.ShapeDtypeStruct((B, H2), jnp.float32),
)


def kernel(x_categorical, x_numerical, tables, bn_gamma, bn_beta, bn_mean, bn_var,
           W1, b1, W2, b2):
    x_cat = x_categorical.astype(jnp.int32)
    flat_idx = (x_cat + (jnp.arange(N_FIELDS, dtype=jnp.int32) * VOCAB)[None, :]
                ).reshape(-1)
    flat_tables = tables.reshape(N_FIELDS * VOCAB, EMB)
    emb = _sc_gather(flat_idx, flat_tables).reshape(B, EMB_DIM)

    scale = bn_gamma * lax.rsqrt(bn_var + 1e-5)
    shift = bn_beta - bn_mean * scale
    W1e_T = W1[:, :EMB_DIM].T
    W1n = W1[:, EMB_DIM:]
    W1n_T = (W1n * scale[None, :]).T
    W1n_T_pad = jnp.zeros((NUM_PAD, H1), jnp.float32).at[:NUM].set(W1n_T)
    b1_eff = (b1 + W1n @ shift).reshape(1, H1)
    x_num_pad = jnp.zeros((B, NUM_PAD), jnp.float32).at[:, :NUM].set(x_numerical)

    return _mlp(emb, x_num_pad, W1e_T, W1n_T_pad, b1_eff, W2.T, b2.reshape(1, H2))
